# baseline (speedup 1.0000x reference)
import functools

import jax
import jax.numpy as jnp
from jax import lax
from jax.experimental import pallas as pl
from jax.experimental.pallas import tpu as pltpu

BATCH = 2
NUM_NEIGHBORS = 9
BIG = 2**30


def _body(e_ref, mb_ref, mbany_ref, pred_ref, ps_ref,
          e2_ref, runmin_ref, runarg_ref, x_ref, d2x_ref, smem_ref, sem,
          *, Q, D, M, BM, NB):
    i = pl.program_id(0)
    NP = Q // BATCH
    dn = (((1,), (1,)), ((), ()))
    prec = lax.Precision.HIGHEST

    @pl.when(i == 0)
    def _init():
        e = e_ref[:, :]
        e2_ref[:, :] = jnp.sum(e * e, axis=1, keepdims=True)
        runmin_ref[:, :] = jnp.full((Q, 1), jnp.inf, jnp.float32)
        runarg_ref[:, :] = jnp.zeros((Q, 1), jnp.int32)

    mb = mb_ref[:, :]
    ones_row = jnp.ones((1, D), jnp.float32)
    y2row = lax.dot_general(ones_row, mb * mb, dn, precision=prec)

    @pl.when(i < NB)
    def _phase1():
        p = lax.dot_general(e_ref[:, :], mb, dn, precision=prec)
        d2 = (e2_ref[:, :] - 2.0 * p) + y2row
        bmin = jnp.min(d2, axis=1, keepdims=True)
        lio = lax.broadcasted_iota(jnp.int32, (Q, BM), 1)
        barg = jnp.min(jnp.where(d2 == bmin, lio, BIG), axis=1,
                       keepdims=True) + i * BM
        better = bmin < runmin_ref[:, :]
        runarg_ref[:, :] = jnp.where(better, barg, runarg_ref[:, :])
        runmin_ref[:, :] = jnp.where(better, bmin, runmin_ref[:, :])

    @pl.when(i == NB - 1)
    def _boundary():
        qio = lax.broadcasted_iota(jnp.int32, (Q, 1), 0)
        rm = runmin_ref[:, :]
        ra = runarg_ref[:, :]
        x_ref[pl.ds(4, 4), :] = jnp.zeros((4, D), jnp.float32)
        for b in range(BATCH):
            mask = (qio >= b * NP) & (qio < (b + 1) * NP)
            mv = jnp.max(jnp.where(mask, rm, -jnp.inf))
            aq = jnp.min(jnp.where(mask & (rm == mv), qio, BIG))
            nn = jnp.sum(jnp.where(qio == aq, ra, 0))
            smem_ref[b] = mv
            x_ref[pl.ds(BATCH + b, 1), :] = e_ref[pl.ds(aq, 1), :]
            cp = pltpu.make_async_copy(mbany_ref.at[pl.ds(nn, 1), :],
                                       x_ref.at[pl.ds(b, 1), :], sem)
            cp.start()
            cp.wait()

    @pl.when(i >= NB)
    def _phase2():
        j = i - NB
        x = x_ref[:, :]
        x2 = jnp.sum(x * x, axis=1, keepdims=True)
        p = lax.dot_general(x, mb, dn, precision=prec)
        d2x_ref[j] = (x2 - 2.0 * p) + y2row

    @pl.when(i == 2 * NB - 1)
    def _final():
        ps_ref[:, :] = jnp.sqrt(jnp.maximum(runmin_ref[:, :], 0.0))
        gio = (lax.broadcasted_iota(jnp.int32, (NB, BM), 0) * BM
               + lax.broadcasted_iota(jnp.int32, (NB, BM), 1))
        tlio = lax.broadcasted_iota(jnp.int32, (1, 16), 1)
        for b in range(BATCH):
            work = d2x_ref[:, b, :]
            rowf = d2x_ref[:, BATCH + b, :]
            dvec = jnp.zeros((1, 16), jnp.float32)
            for t in range(NUM_NEIGHBORS):
                mv = jnp.min(work)
                ai = jnp.min(jnp.where(work == mv, gio, BIG))
                sel = gio == ai
                dval = jnp.sum(jnp.where(sel, rowf, 0.0))
                dvec = jnp.where(tlio == t,
                                 jnp.sqrt(jnp.maximum(dval, 0.0)), dvec)
                work = jnp.where(sel, jnp.inf, work)
            valid = tlio < NUM_NEIGHBORS
            mx = jnp.max(jnp.where(valid, dvec, -jnp.inf))
            ev = jnp.where(valid, jnp.exp(dvec - mx), 0.0)
            s0 = jnp.sum(jnp.where(tlio == 0, ev, 0.0))
            w = 1.0 - s0 / jnp.sum(ev)
            score = jnp.sqrt(jnp.maximum(smem_ref[b], 0.0))
            pred_ref[pl.ds(b, 1), :] = jnp.full((1, 1), w * score, jnp.float32)


@jax.jit
def kernel(embedding, memory_bank):
    Q, D = embedding.shape
    M = memory_bank.shape[0]
    BM = 1024
    NB = M // BM
    body = functools.partial(_body, Q=Q, D=D, M=M, BM=BM, NB=NB)
    pred, ps = pl.pallas_call(
        body,
        grid=(2 * NB,),
        in_specs=[
            pl.BlockSpec((Q, D), lambda i: (0, 0)),
            pl.BlockSpec((BM, D), lambda i: (i % NB, 0)),
            pl.BlockSpec(memory_space=pl.ANY),
        ],
        out_specs=[
            pl.BlockSpec((BATCH, 1), lambda i: (0, 0)),
            pl.BlockSpec((Q, 1), lambda i: (0, 0)),
        ],
        out_shape=[
            jax.ShapeDtypeStruct((BATCH, 1), jnp.float32),
            jax.ShapeDtypeStruct((Q, 1), jnp.float32),
        ],
        scratch_shapes=[
            pltpu.VMEM((Q, 1), jnp.float32),
            pltpu.VMEM((Q, 1), jnp.float32),
            pltpu.VMEM((Q, 1), jnp.int32),
            pltpu.VMEM((8, D), jnp.float32),
            pltpu.VMEM((NB, 8, BM), jnp.float32),
            pltpu.SMEM((BATCH,), jnp.float32),
            pltpu.SemaphoreType.DMA,
        ],
    )(embedding, memory_bank, memory_bank)
    H = 28
    W = (Q // BATCH) // H
    return pred.reshape(BATCH), ps.reshape(BATCH, 1, W, H)


# speedup vs baseline: 2.8841x; 2.8841x over previous
import functools

import jax
import jax.numpy as jnp
from jax import lax
from jax.experimental import pallas as pl
from jax.experimental.pallas import tpu as pltpu

BATCH = 2
NUM_NEIGHBORS = 9
BIG = 2**30


def _body(e_ref, mb_ref, mbany_ref, pred_ref, ps_ref,
          eb_ref, e2_ref, runmin_ref, x_ref, d2f_ref, d2n_ref,
          smem_ref, sem, *, Q, D, M, BM, NB):
    i = pl.program_id(0)
    NP = Q // BATCH
    dn = (((1,), (1,)), ((), ()))
    hi = lax.Precision.HIGHEST

    @pl.when(i == 0)
    def _init():
        e = e_ref[:, :]
        eb_ref[:, :] = (-2.0 * e).astype(jnp.bfloat16)
        e2_ref[:, :] = jnp.sum(e * e, axis=1, keepdims=True)
        runmin_ref[:, :] = jnp.full((Q, 1), jnp.inf, jnp.float32)

    mb = mb_ref[:, :]
    mbb = mb.astype(jnp.bfloat16)
    ones_row = jnp.ones((1, D), jnp.float32)
    y2row = lax.dot_general(ones_row, mb * mb, dn, precision=hi)

    @pl.when(i < NB)
    def _phase1():
        p = lax.dot_general(eb_ref[:, :], mbb, dn,
                            preferred_element_type=jnp.float32)
        bmin = jnp.min(p + y2row, axis=1, keepdims=True)
        runmin_ref[:, :] = jnp.minimum(runmin_ref[:, :], bmin)

    @pl.when(i == NB - 1)
    def _argmax_patches():
        qio = lax.broadcasted_iota(jnp.int32, (Q, 1), 0)
        s = runmin_ref[:, :] + e2_ref[:, :]
        x_ref[pl.ds(2, 6), :] = jnp.zeros((6, D), jnp.float32)
        for b in range(BATCH):
            mask = (qio >= b * NP) & (qio < (b + 1) * NP)
            mv = jnp.max(jnp.where(mask, s, -jnp.inf))
            aq = jnp.min(jnp.where(mask & (s == mv), qio, BIG))
            smem_ref[b] = mv
            x_ref[pl.ds(b, 1), :] = e_ref[pl.ds(aq, 1), :]

    def _xscan(out_ref, j):
        x = x_ref[:, :]
        x2 = jnp.sum(x * x, axis=1, keepdims=True)
        p = lax.dot_general(x.astype(jnp.bfloat16), mbb, dn,
                            preferred_element_type=jnp.float32)
        out_ref[j] = (x2 - 2.0 * p) + y2row

    @pl.when((i >= NB) & (i < 2 * NB))
    def _phase2a():
        _xscan(d2f_ref, i - NB)

    @pl.when(i == 2 * NB - 1)
    def _gather_nn():
        gio = (lax.broadcasted_iota(jnp.int32, (NB, BM), 0) * BM
               + lax.broadcasted_iota(jnp.int32, (NB, BM), 1))
        for b in range(BATCH):
            row = d2f_ref[:, b, :]
            mv = jnp.min(row)
            nn = jnp.min(jnp.where(row == mv, gio, BIG))
            cp = pltpu.make_async_copy(mbany_ref.at[pl.ds(nn, 1), :],
                                       x_ref.at[pl.ds(BATCH + b, 1), :], sem)
            cp.start()
            cp.wait()

    @pl.when(i >= 2 * NB)
    def _phase2b():
        _xscan(d2n_ref, i - 2 * NB)

    @pl.when(i == 3 * NB - 1)
    def _final():
        ps_ref[:, :] = jnp.sqrt(jnp.maximum(
            runmin_ref[:, :] + e2_ref[:, :], 0.0))
        gio = (lax.broadcasted_iota(jnp.int32, (NB, BM), 0) * BM
               + lax.broadcasted_iota(jnp.int32, (NB, BM), 1))
        tlio = lax.broadcasted_iota(jnp.int32, (1, 16), 1)
        for b in range(BATCH):
            work = d2n_ref[:, BATCH + b, :]
            rowf = d2f_ref[:, b, :]
            dvec = jnp.zeros((1, 16), jnp.float32)
            for t in range(NUM_NEIGHBORS):
                mv = jnp.min(work)
                ai = jnp.min(jnp.where(work == mv, gio, BIG))
                sel = gio == ai
                dval = jnp.sum(jnp.where(sel, rowf, 0.0))
                dvec = jnp.where(tlio == t,
                                 jnp.sqrt(jnp.maximum(dval, 0.0)), dvec)
                work = jnp.where(sel, jnp.inf, work)
            valid = tlio < NUM_NEIGHBORS
            mx = jnp.max(jnp.where(valid, dvec, -jnp.inf))
            ev = jnp.where(valid, jnp.exp(dvec - mx), 0.0)
            s0 = jnp.sum(jnp.where(tlio == 0, ev, 0.0))
            w = 1.0 - s0 / jnp.sum(ev)
            score = jnp.sqrt(jnp.maximum(smem_ref[b], 0.0))
            pred_ref[pl.ds(b, 1), :] = jnp.full((1, 1), w * score, jnp.float32)


@jax.jit
def kernel(embedding, memory_bank):
    Q, D = embedding.shape
    M = memory_bank.shape[0]
    BM = 2048
    NB = M // BM
    body = functools.partial(_body, Q=Q, D=D, M=M, BM=BM, NB=NB)
    pred, ps = pl.pallas_call(
        body,
        grid=(3 * NB,),
        in_specs=[
            pl.BlockSpec((Q, D), lambda i: (0, 0)),
            pl.BlockSpec((BM, D), lambda i: (i % NB, 0)),
            pl.BlockSpec(memory_space=pl.ANY),
        ],
        out_specs=[
            pl.BlockSpec((BATCH, 1), lambda i: (0, 0)),
            pl.BlockSpec((Q, 1), lambda i: (0, 0)),
        ],
        out_shape=[
            jax.ShapeDtypeStruct((BATCH, 1), jnp.float32),
            jax.ShapeDtypeStruct((Q, 1), jnp.float32),
        ],
        scratch_shapes=[
            pltpu.VMEM((Q, D), jnp.bfloat16),
            pltpu.VMEM((Q, 1), jnp.float32),
            pltpu.VMEM((Q, 1), jnp.float32),
            pltpu.VMEM((8, D), jnp.float32),
            pltpu.VMEM((NB, 8, BM), jnp.float32),
            pltpu.VMEM((NB, 8, BM), jnp.float32),
            pltpu.SMEM((BATCH,), jnp.float32),
            pltpu.SemaphoreType.DMA,
        ],
    )(embedding, memory_bank, memory_bank)
    H = 28
    W = (Q // BATCH) // H
    return pred.reshape(BATCH), ps.reshape(BATCH, 1, W, H)


# speedup vs baseline: 4.7276x; 1.6392x over previous
import functools

import jax
import jax.numpy as jnp
from jax import lax
from jax.experimental import pallas as pl
from jax.experimental.pallas import tpu as pltpu

BATCH = 2
NUM_NEIGHBORS = 9
BIG = 2**30
KAUG = 256


def _body(e_ref, mb_ref, mbany_ref, pred_ref, ps_ref,
          ebaug_ref, e2_ref, runmin_ref, runblk_ref, mblk_ref, xrow_ref,
          xaug_ref, d2x_ref, supp_ref, smemf_ref, smemi_ref, sem,
          *, Q, D, M, BM, NB):
    i = pl.program_id(0)
    NP = Q // BATCH
    dn = (((1,), (1,)), ((), ()))

    def _aug(mb):
        n = mb.shape[0]
        y2 = jnp.sum(mb * mb, axis=1, keepdims=True)
        h1 = y2.astype(jnp.bfloat16)
        r1 = y2 - h1.astype(jnp.float32)
        h2 = r1.astype(jnp.bfloat16)
        r2 = r1 - h2.astype(jnp.float32)
        h3 = r2.astype(jnp.bfloat16)
        return jnp.concatenate(
            [mb.astype(jnp.bfloat16), h1, h2, h3,
             jnp.zeros((n, KAUG - D - 3), jnp.bfloat16)], axis=1)

    @pl.when(i == 0)
    def _init():
        e = e_ref[:, :]
        ebaug_ref[:, :] = jnp.concatenate(
            [(-2.0 * e).astype(jnp.bfloat16),
             jnp.ones((Q, 3), jnp.bfloat16),
             jnp.zeros((Q, KAUG - D - 3), jnp.bfloat16)], axis=1)
        e2_ref[:, :] = jnp.sum(e * e, axis=1, keepdims=True)
        runmin_ref[:, :] = jnp.full((Q, 1), jnp.inf, jnp.float32)
        runblk_ref[:, :] = jnp.zeros((Q, 1), jnp.int32)
        xrow_ref[:, :] = jnp.zeros((8, D), jnp.float32)
        supp_ref[:, :] = jnp.zeros((32, D), jnp.float32)

    @pl.when(i < NB)
    def _phase1():
        aug = _aug(mb_ref[:, :])
        mm = lax.dot_general(ebaug_ref[:, :], aug, dn,
                             preferred_element_type=jnp.float32)
        bmin = jnp.min(mm, axis=1, keepdims=True)
        better = bmin < runmin_ref[:, :]
        runblk_ref[:, :] = jnp.where(better, i, runblk_ref[:, :])
        runmin_ref[:, :] = jnp.where(better, bmin, runmin_ref[:, :])

    @pl.when(i == NB - 1)
    def _boundary():
        qio = lax.broadcasted_iota(jnp.int32, (Q, 1), 0)
        lio = lax.broadcasted_iota(jnp.int32, (1, BM), 1)
        s = runmin_ref[:, :] + e2_ref[:, :]
        for b in range(BATCH):
            mask = (qio >= b * NP) & (qio < (b + 1) * NP)
            mv = jnp.max(jnp.where(mask, s, -jnp.inf))
            aq = jnp.min(jnp.where(mask & (s == mv), qio, BIG))
            blk = jnp.sum(jnp.where(qio == aq, runblk_ref[:, :], 0))
            smemf_ref[b] = mv
            smemi_ref[b] = aq
            cp = pltpu.make_async_copy(
                mbany_ref.at[pl.ds(blk * BM, BM), :], mblk_ref, sem)
            cp.start()
            cp.wait()
            erow = jnp.concatenate(
                [(-2.0 * e_ref[pl.ds(aq, 1), :]).astype(jnp.bfloat16),
                 jnp.ones((1, 3), jnp.bfloat16),
                 jnp.zeros((1, KAUG - D - 3), jnp.bfloat16)], axis=1)
            row = lax.dot_general(erow, _aug(mblk_ref[:, :]), dn,
                                  preferred_element_type=jnp.float32)
            rmin = jnp.min(row)
            nn = blk * BM + jnp.min(jnp.where(row == rmin, lio, BIG))
            cp2 = pltpu.make_async_copy(
                mbany_ref.at[pl.ds(nn, 1), :],
                xrow_ref.at[pl.ds(b, 1), :], sem)
            cp2.start()
            cp2.wait()
        xaug_ref[:, :] = jnp.concatenate(
            [(-2.0 * xrow_ref[:, :]).astype(jnp.bfloat16),
             jnp.ones((8, 3), jnp.bfloat16),
             jnp.zeros((8, KAUG - D - 3), jnp.bfloat16)], axis=1)

    @pl.when(i >= NB)
    def _phase2():
        mm2 = lax.dot_general(xaug_ref[:, :], _aug(mb_ref[:, :]), dn,
                              preferred_element_type=jnp.float32)
        d2x_ref[i - NB] = mm2

    @pl.when(i == 2 * NB - 1)
    def _final():
        ps_ref[:, :] = jnp.sqrt(jnp.maximum(
            runmin_ref[:, :] + e2_ref[:, :], 0.0))
        gio = (lax.broadcasted_iota(jnp.int32, (NB, BM), 0) * BM
               + lax.broadcasted_iota(jnp.int32, (NB, BM), 1))
        sio = lax.broadcasted_iota(jnp.int32, (16, 1), 0)
        for b in range(BATCH):
            work = d2x_ref[:, b, :]
            cps = []
            for t in range(NUM_NEIGHBORS):
                mv = jnp.min(work)
                ai = jnp.min(jnp.where(work == mv, gio, BIG))
                cp = pltpu.make_async_copy(
                    mbany_ref.at[pl.ds(ai, 1), :],
                    supp_ref.at[pl.ds(b * 16 + t, 1), :], sem)
                cp.start()
                cps.append(cp)
                work = jnp.where(gio == ai, jnp.inf, work)
            for cp in cps:
                cp.wait()
        for b in range(BATCH):
            f = e_ref[pl.ds(smemi_ref[b], 1), :]
            sf = supp_ref[pl.ds(b * 16, 16), :]
            diff = sf - f
            dist = jnp.sqrt(jnp.maximum(
                jnp.sum(diff * diff, axis=1, keepdims=True), 0.0))
            valid = sio < NUM_NEIGHBORS
            mx = jnp.max(jnp.where(valid, dist, -jnp.inf))
            ev = jnp.where(valid, jnp.exp(dist - mx), 0.0)
            s0 = jnp.sum(jnp.where(sio == 0, ev, 0.0))
            w = 1.0 - s0 / jnp.sum(ev)
            score = jnp.sqrt(jnp.maximum(smemf_ref[b], 0.0))
            pred_ref[pl.ds(b, 1), :] = jnp.full((1, 1), w * score,
                                                jnp.float32)


@jax.jit
def kernel(embedding, memory_bank):
    Q, D = embedding.shape
    M = memory_bank.shape[0]
    BM = 2048
    NB = M // BM
    body = functools.partial(_body, Q=Q, D=D, M=M, BM=BM, NB=NB)
    pred, ps = pl.pallas_call(
        body,
        grid=(2 * NB,),
        in_specs=[
            pl.BlockSpec((Q, D), lambda i: (0, 0)),
            pl.BlockSpec((BM, D), lambda i: (i % NB, 0)),
            pl.BlockSpec(memory_space=pl.ANY),
        ],
        out_specs=[
            pl.BlockSpec((BATCH, 1), lambda i: (0, 0)),
            pl.BlockSpec((Q, 1), lambda i: (0, 0)),
        ],
        out_shape=[
            jax.ShapeDtypeStruct((BATCH, 1), jnp.float32),
            jax.ShapeDtypeStruct((Q, 1), jnp.float32),
        ],
        scratch_shapes=[
            pltpu.VMEM((Q, KAUG), jnp.bfloat16),
            pltpu.VMEM((Q, 1), jnp.float32),
            pltpu.VMEM((Q, 1), jnp.float32),
            pltpu.VMEM((Q, 1), jnp.int32),
            pltpu.VMEM((BM, D), jnp.float32),
            pltpu.VMEM((8, D), jnp.float32),
            pltpu.VMEM((8, KAUG), jnp.bfloat16),
            pltpu.VMEM((NB, 8, BM), jnp.float32),
            pltpu.VMEM((32, D), jnp.float32),
            pltpu.SMEM((BATCH,), jnp.float32),
            pltpu.SMEM((BATCH,), jnp.int32),
            pltpu.SemaphoreType.DMA,
        ],
    )(embedding, memory_bank, memory_bank)
    H = 28
    W = (Q // BATCH) // H
    return pred.reshape(BATCH), ps.reshape(BATCH, 1, W, H)
